# baseline (speedup 1.0000x reference)
import functools

import jax
import jax.numpy as jnp
import numpy as np
from jax import lax
from jax.experimental import pallas as pl
from jax.experimental.pallas import tpu as pltpu
from jax.experimental.pallas import tpu_sc as plsc

B = 4096
V = 100000
T = 200
NW = 32
CHUNK = (B * T) // NW
ROWS_PER_W = B // NW
LANES = 16
NCHUNKS = CHUNK // LANES
SW = 128
NSTR = CHUNK // SW
IN_FLIGHT = 16
SCALE = -1.0 / float(B * T)

_ROWOFF = np.repeat(np.arange(ROWS_PER_W, dtype=np.int32) * V, T)


def _mesh():
    return plsc.VectorSubcoreMesh(core_axis_name="c", subcore_axis_name="s")


@functools.partial(
    pl.kernel,
    mesh=_mesh(),
    compiler_params=pltpu.CompilerParams(use_tc_tiling_on_sc=True),
    out_type=jax.ShapeDtypeStruct((NW, LANES), jnp.float32),
    scratch_types=[
        pltpu.VMEM((CHUNK,), jnp.int32),
        pltpu.VMEM((CHUNK,), jnp.int32),
        pltpu.VMEM((CHUNK,), jnp.float32),
        pltpu.VMEM((LANES,), jnp.float32),
        pltpu.SemaphoreType.DMA,
    ],
)
def _sc_gather_loss(rowoff_hbm, flat_hbm, tags_hbm, out_hbm,
                    lbl_v, idx_v, val_v, part_v, sem):
    nc = 2
    wid = lax.axis_index("s") * nc + lax.axis_index("c")
    base = wid * CHUNK


    row_base = wid * (ROWS_PER_W * V)

    def ixbody(i, c):
        sl = pl.ds(i * LANES, LANES)
        idx_v[sl] = idx_v[sl] + lbl_v[sl] + row_base
        return c


    def _copy(j):
        sl = pl.ds(j * SW, SW)
        return pltpu.make_async_copy(flat_hbm.at[idx_v.at[sl]],
                                     val_v.at[sl], sem)

    def fire(j, c):
        _copy(j).start()
        return c

    def fire_wait(j, c):
        _copy(j).wait()
        _copy(j + IN_FLIGHT).start()
        return c

    def drain(j, c):
        _copy(j).wait()
        return c


    def rbody(i, acc):
        sl = pl.ds(i * LANES, LANES)
        v = val_v[sl]
        m = lbl_v[sl] != 0
        return acc + jnp.where(m, v, 0.0)

    acc = jnp.zeros((LANES,), jnp.float32)

    part_v[...] = acc
    pltpu.sync_copy(part_v, out_hbm.at[wid])


def kernel(log_prob, tags_label):
    flat = log_prob
    tags = tags_label.reshape(-1).astype(jnp.int32)
    rowoff = jnp.asarray(_ROWOFF)
    partials = _sc_gather_loss(rowoff, flat, tags)
    return jnp.sum(partials) * SCALE


# speedup vs baseline: 16.5029x; 16.5029x over previous
import functools

import jax
import jax.numpy as jnp
import numpy as np
from jax import lax
from jax.experimental import pallas as pl
from jax.experimental.pallas import tpu as pltpu
from jax.experimental.pallas import tpu_sc as plsc

B = 4096
V = 100000
T = 200
NW = 32
CHUNK = (B * T) // NW
ROWS_PER_W = B // NW
LANES = 16
NCHUNKS = CHUNK // LANES
SW = 128
NSTR = CHUNK // SW
IN_FLIGHT = 16
SCALE = -1.0 / float(B * T)

_RQ = np.repeat(np.arange(ROWS_PER_W, dtype=np.int32), T)


def _mesh():
    return plsc.VectorSubcoreMesh(core_axis_name="c", subcore_axis_name="s")


@functools.partial(
    pl.kernel,
    mesh=_mesh(),
    out_type=jax.ShapeDtypeStruct((NW, LANES), jnp.float32),
    scratch_types=[
        pltpu.VMEM((CHUNK,), jnp.int32),
        pltpu.VMEM((CHUNK,), jnp.int32),
        pltpu.VMEM((CHUNK,), jnp.float32),
        pltpu.VMEM((LANES,), jnp.float32),
        pltpu.SemaphoreType.DMA,
    ],
)
def _sc_gather_loss(rq_hbm, flat_hbm, tags_hbm, out_hbm,
                    lbl_v, idx_v, val_v, part_v, sem):
    nc = 2
    wid = lax.axis_index("s") * nc + lax.axis_index("c")
    base = wid * CHUNK

    pltpu.sync_copy(tags_hbm.at[pl.ds(base, CHUNK)], lbl_v)
    pltpu.sync_copy(rq_hbm, idx_v)

    wbase = wid * 1024

    def ixbody(i, c):
        sl = pl.ds(i * LANES, LANES)
        v = lbl_v[sl]
        idx_v[sl] = ((v >> 3) << 15) + ((v & 7) << 7) + (idx_v[sl] + wbase)
        return c

    lax.fori_loop(0, NCHUNKS, ixbody, 0, unroll=4)

    def _copy(j):
        sl = pl.ds(j * SW, SW)
        return pltpu.make_async_copy(flat_hbm.at[idx_v.at[sl]],
                                     val_v.at[sl], sem)

    def fire(j, c):
        _copy(j).start()
        return c

    def fire_wait(j, c):
        _copy(j).wait()
        _copy(j + IN_FLIGHT).start()
        return c

    def drain(j, c):
        _copy(j).wait()
        return c

    lax.fori_loop(0, IN_FLIGHT, fire, 0)
    lax.fori_loop(0, NSTR - IN_FLIGHT, fire_wait, 0)
    lax.fori_loop(NSTR - IN_FLIGHT, NSTR, drain, 0)

    def rbody(i, acc):
        sl = pl.ds(i * LANES, LANES)
        v = val_v[sl]
        m = lbl_v[sl] != 0
        return acc + jnp.where(m, v, 0.0)

    acc = lax.fori_loop(0, NCHUNKS, rbody, jnp.zeros((LANES,), jnp.float32),
                        unroll=4)

    part_v[...] = acc * SCALE
    pltpu.sync_copy(part_v, out_hbm.at[wid])


def kernel(log_prob, tags_label):
    flat = (log_prob.T
            .reshape(V // 8, 8, B // 128, 128)
            .transpose(0, 2, 1, 3)
            .reshape(-1))
    tags = tags_label.reshape(-1).astype(jnp.int32)
    rq = jnp.asarray(_RQ)
    partials = _sc_gather_loss(rq, flat, tags)
    return jnp.sum(partials)


# speedup vs baseline: 24.4115x; 1.4792x over previous
import functools

import jax
import jax.numpy as jnp
from jax import lax
from jax.experimental import pallas as pl
from jax.experimental.pallas import tpu as pltpu
from jax.experimental.pallas import tpu_sc as plsc

B = 4096
V = 100000
T = 200
NW = 32
CHUNK = (B * T) // NW
LANES = 16
SW = 128
G = 20
NG = CHUNK // (G * SW)
GEL = G * SW
GCH = GEL // LANES
SCALE = -1.0 / float(B * T)


def _mesh():
    return plsc.VectorSubcoreMesh(core_axis_name="c", subcore_axis_name="s")


@functools.partial(
    pl.kernel,
    mesh=_mesh(),
    out_type=jax.ShapeDtypeStruct((NW, LANES), jnp.float32),
    scratch_types=[
        pltpu.VMEM((CHUNK,), jnp.int32),
        pltpu.VMEM((CHUNK,), jnp.int32),
        pltpu.VMEM((CHUNK,), jnp.float32),
        pltpu.VMEM((LANES,), jnp.float32),
        pltpu.SemaphoreType.DMA,
        pltpu.SemaphoreType.DMA,
        pltpu.SemaphoreType.DMA,
    ],
)
def _sc_gather_loss(tags_hbm, flat_hbm, out_hbm,
                    lbl_v, idx_v, val_v, part_v, sem_a, sem_b, sem_l):
    nc = 2
    wid = lax.axis_index("s") * nc + lax.axis_index("c")

    def lstage(m, c):
        pltpu.make_async_copy(tags_hbm.at[m, wid], lbl_v.at[pl.ds(m * 1024, 1024)],
                              sem_l).start()
        return c

    lax.fori_loop(0, T // 8, lstage, 0)

    def lwait(m, c):
        pltpu.make_async_copy(tags_hbm.at[0, 0], lbl_v.at[pl.ds(0, 1024)],
                              sem_l).wait()
        return c

    lax.fori_loop(0, T // 8, lwait, 0)

    wbase = wid * 1024
    lane = lax.iota(jnp.int32, 16)

    def cbody(i, c):
        sl = pl.ds(i * LANES, LANES)
        v = lbl_v[sl]
        q0 = (i & 7) << 4
        idx_v[sl] = ((v >> 3) << 15) + ((v & 7) << 7) + (wbase + q0 + lane)
        return c

    def compute(g):
        lax.fori_loop(g * GCH, (g + 1) * GCH, cbody, 0, unroll=4)

    def fire(g, sem):
        def fbody(t, c):
            sl = pl.ds(t * SW, SW)
            pltpu.make_async_copy(flat_hbm.at[idx_v.at[sl]], val_v.at[sl],
                                  sem).start()
            return c
        lax.fori_loop(g * G, (g + 1) * G, fbody, 0)

    def drain(g, sem):
        sl = pl.ds(g * GEL, GEL)
        pltpu.make_async_copy(flat_hbm.at[idx_v.at[sl]], val_v.at[sl],
                              sem).wait()

    def rbody(i, acc):
        sl = pl.ds(i * LANES, LANES)
        return acc + jnp.where(lbl_v[sl] != 0, val_v[sl], 0.0)

    def reduce(g, acc):
        return lax.fori_loop(g * GCH, (g + 1) * GCH, rbody, acc, unroll=4)

    sems = (sem_a, sem_b)
    compute(0)
    fire(0, sems[0])
    acc = jnp.zeros((LANES,), jnp.float32)
    for g in range(NG):
        if g + 1 < NG:
            compute(g + 1)
            fire(g + 1, sems[(g + 1) % 2])
        drain(g, sems[g % 2])
        acc = reduce(g, acc)

    part_v[...] = acc * SCALE
    pltpu.sync_copy(part_v, out_hbm.at[wid])


def kernel(log_prob, tags_label):
    flat = (log_prob.T
            .reshape(V // 8, 8, B // 128, 128)
            .transpose(0, 2, 1, 3)
            .reshape(-1))
    tags = (tags_label.astype(jnp.int32).T
            .reshape(T // 8, 8, B // 128, 128)
            .transpose(0, 2, 1, 3)
            .reshape(T // 8, B // 128, 1024))
    partials = _sc_gather_loss(tags, flat)
    return jnp.sum(partials)


# speedup vs baseline: 24.5803x; 1.0069x over previous
import functools

import jax
import jax.numpy as jnp
from jax import lax
from jax.experimental import pallas as pl
from jax.experimental.pallas import tpu as pltpu
from jax.experimental.pallas import tpu_sc as plsc

B = 4096
V = 100000
T = 200
NW = 32
CHUNK = (B * T) // NW
LANES = 16
SW = 128
G = 20
NG = CHUNK // (G * SW)
GEL = G * SW
GCH = GEL // LANES
SCALE = -1.0 / float(B * T)


def _mesh():
    return plsc.VectorSubcoreMesh(core_axis_name="c", subcore_axis_name="s")


@functools.partial(
    pl.kernel,
    mesh=_mesh(),
    out_type=jax.ShapeDtypeStruct((NW, LANES), jnp.float32),
    scratch_types=[
        pltpu.VMEM((CHUNK // SW, SW), jnp.int32),
        pltpu.VMEM((CHUNK,), jnp.int32),
        pltpu.VMEM((CHUNK,), jnp.float32),
        pltpu.VMEM((LANES,), jnp.float32),
        pltpu.SemaphoreType.DMA,
        pltpu.SemaphoreType.DMA,
        pltpu.SemaphoreType.DMA,
    ],
)
def _sc_gather_loss(tags_hbm, flat_hbm, out_hbm,
                    lbl_v, idx_v, val_v, part_v, sem_a, sem_b, sem_l):
    nc = 2
    wid = lax.axis_index("s") * nc + lax.axis_index("c")

    def lstage(m, c):
        pltpu.make_async_copy(tags_hbm.at[m, wid],
                              lbl_v.at[pl.ds(m * 8, 8), :], sem_l).start()
        return c

    lax.fori_loop(0, T // 8, lstage, 0)

    def lwait(m, c):
        pltpu.make_async_copy(tags_hbm.at[0, 0],
                              lbl_v.at[pl.ds(0, 8), :], sem_l).wait()
        return c

    lax.fori_loop(0, T // 8, lwait, 0)

    wbase = wid * 1024
    lane = lax.iota(jnp.int32, 16)

    def cbody(i, c):
        sl = pl.ds(i * LANES, LANES)
        q0 = (i & 7) << 4
        v = lbl_v[i >> 3, pl.ds(q0, LANES)]
        idx_v[sl] = ((v >> 3) << 15) + ((v & 7) << 7) + (wbase + q0 + lane)
        return c

    def compute(g):
        lax.fori_loop(g * GCH, (g + 1) * GCH, cbody, 0, unroll=4)

    def fire(g, sem):
        def fbody(t, c):
            sl = pl.ds(t * SW, SW)
            pltpu.make_async_copy(flat_hbm.at[idx_v.at[sl]], val_v.at[sl],
                                  sem).start()
            return c
        lax.fori_loop(g * G, (g + 1) * G, fbody, 0)

    def drain(g, sem):
        sl = pl.ds(g * GEL, GEL)
        pltpu.make_async_copy(flat_hbm.at[idx_v.at[sl]], val_v.at[sl],
                              sem).wait()

    def rbody(i, acc):
        sl = pl.ds(i * LANES, LANES)
        v = lbl_v[i >> 3, pl.ds((i & 7) << 4, LANES)]
        return acc + jnp.where(v != 0, val_v[sl], 0.0)

    def reduce(g, acc):
        return lax.fori_loop(g * GCH, (g + 1) * GCH, rbody, acc, unroll=4)

    sems = (sem_a, sem_b)
    compute(0)
    fire(0, sems[0])
    acc = jnp.zeros((LANES,), jnp.float32)
    for g in range(NG):
        if g + 1 < NG:
            compute(g + 1)
            fire(g + 1, sems[(g + 1) % 2])
        drain(g, sems[g % 2])
        acc = reduce(g, acc)

    part_v[...] = acc * SCALE
    pltpu.sync_copy(part_v, out_hbm.at[wid])


def kernel(log_prob, tags_label):
    flat = (log_prob.T
            .reshape(V // 8, 8, B // 128, 128)
            .transpose(0, 2, 1, 3)
            .reshape(-1))
    tags = (tags_label.astype(jnp.int32).T
            .reshape(T // 8, 8, B // 128, 128)
            .transpose(0, 2, 1, 3))
    partials = _sc_gather_loss(tags, flat)
    return jnp.sum(partials)


# speedup vs baseline: 24.8528x; 1.0111x over previous
import functools

import jax
import jax.numpy as jnp
from jax import lax
from jax.experimental import pallas as pl
from jax.experimental.pallas import tpu as pltpu
from jax.experimental.pallas import tpu_sc as plsc

B = 4096
V = 100000
T = 200
NW = 32
CHUNK = (B * T) // NW
LANES = 16
SW = 128
G = 20
NG = CHUNK // (G * SW)
GEL = G * SW
GCH = GEL // LANES
SCALE = -1.0 / float(B * T)


def _mesh():
    return plsc.VectorSubcoreMesh(core_axis_name="c", subcore_axis_name="s")


@functools.partial(
    pl.kernel,
    mesh=_mesh(),
    out_type=jax.ShapeDtypeStruct((NW, LANES), jnp.float32),
    scratch_types=[
        pltpu.VMEM((CHUNK // SW, SW), jnp.int32),
        pltpu.VMEM((CHUNK,), jnp.int32),
        pltpu.VMEM((CHUNK,), jnp.float32),
        pltpu.VMEM((LANES,), jnp.float32),
        pltpu.SemaphoreType.DMA,
        pltpu.SemaphoreType.DMA,
        pltpu.SemaphoreType.DMA,
    ],
)
def _sc_gather_loss(tags_hbm, flat_hbm, out_hbm,
                    lbl_v, idx_v, val_v, part_v, sem_a, sem_b, sem_l):
    nc = 2
    wid = lax.axis_index("s") * nc + lax.axis_index("c")

    def lstage(m, c):
        pltpu.make_async_copy(tags_hbm.at[m, wid],
                              lbl_v.at[pl.ds(m * 8, 8), :], sem_l).start()
        return c

    lax.fori_loop(0, T // 8, lstage, 0)

    def lwait(m, c):
        pltpu.make_async_copy(tags_hbm.at[0, 0],
                              lbl_v.at[pl.ds(0, 8), :], sem_l).wait()
        return c

    lax.fori_loop(0, T // 8, lwait, 0)

    wbase = wid * 1024
    lane = lax.iota(jnp.int32, 16)

    def cbody(i, c):
        sl = pl.ds(i * LANES, LANES)
        q0 = (i & 7) << 4
        v = lbl_v[i >> 3, pl.ds(q0, LANES)]
        idx_v[sl] = ((v >> 3) << 15) + ((v & 7) << 7) + (wbase + q0 + lane)
        return c

    def compute(g):
        lax.fori_loop(g * GCH, (g + 1) * GCH, cbody, 0, unroll=4)

    def fire(g, sem):
        def fbody(t, c):
            sl = pl.ds(t * SW, SW)
            pltpu.make_async_copy(flat_hbm.at[idx_v.at[sl]], val_v.at[sl],
                                  sem).start()
            return c
        lax.fori_loop(g * G, (g + 1) * G, fbody, 0)

    def drain(g, sem):
        sl = pl.ds(g * GEL, GEL)
        pltpu.make_async_copy(flat_hbm.at[idx_v.at[sl]], val_v.at[sl],
                              sem).wait()

    def rbody(i, acc):
        sl = pl.ds(i * LANES, LANES)
        v = lbl_v[i >> 3, pl.ds((i & 7) << 4, LANES)]
        return acc + jnp.where(v != 0, val_v[sl], 0.0)

    def reduce(g, acc):
        return lax.fori_loop(g * GCH, (g + 1) * GCH, rbody, acc, unroll=4)

    def fused_body(gr, gc):
        def fbody(i, acc):
            slr = pl.ds(gr * GEL + i * LANES, LANES)
            ir = gr * GCH + i
            vr = lbl_v[ir >> 3, pl.ds((ir & 7) << 4, LANES)]
            acc = acc + jnp.where(vr != 0, val_v[slr], 0.0)
            ic = gc * GCH + i
            slc = pl.ds(gc * GEL + i * LANES, LANES)
            q0 = (ic & 7) << 4
            v = lbl_v[ic >> 3, pl.ds(q0, LANES)]
            idx_v[slc] = ((v >> 3) << 15) + ((v & 7) << 7) + (wbase + q0 + lane)
            return acc
        return fbody

    sems = (sem_a, sem_b, sem_l)
    compute(0)
    fire(0, sems[0])
    compute(1)
    fire(1, sems[1])
    acc = jnp.zeros((LANES,), jnp.float32)
    for g in range(NG):
        drain(g, sems[g % 3])
        if g + 2 < NG:
            acc = lax.fori_loop(0, GCH, fused_body(g, g + 2), acc, unroll=4)
            fire(g + 2, sems[(g + 2) % 3])
        else:
            acc = reduce(g, acc)

    part_v[...] = acc * SCALE
    pltpu.sync_copy(part_v, out_hbm.at[wid])


def kernel(log_prob, tags_label):
    flat = (log_prob.T
            .reshape(V // 8, 8, B // 128, 128)
            .transpose(0, 2, 1, 3)
            .reshape(-1))
    tags = (tags_label.astype(jnp.int32).T
            .reshape(T // 8, 8, B // 128, 128)
            .transpose(0, 2, 1, 3))
    partials = _sc_gather_loss(tags, flat)
    return jnp.sum(partials)


# speedup vs baseline: 25.6173x; 1.0308x over previous
import functools

import jax
import jax.numpy as jnp
from jax import lax
from jax.experimental import pallas as pl
from jax.experimental.pallas import tpu as pltpu
from jax.experimental.pallas import tpu_sc as plsc

B = 4096
V = 100000
T = 200
NW = 32
CHUNK = (B * T) // NW
LANES = 16
SW = 128
G = 20
NG = CHUNK // (G * SW)
GEL = G * SW
GCH = GEL // LANES
SCALE = -1.0 / float(B * T)


def _mesh():
    return plsc.VectorSubcoreMesh(core_axis_name="c", subcore_axis_name="s")


@functools.partial(
    pl.kernel,
    mesh=_mesh(),
    out_type=jax.ShapeDtypeStruct((NW, LANES), jnp.float32),
    scratch_types=[
        pltpu.VMEM((CHUNK // SW, SW), jnp.int32),
        pltpu.VMEM((CHUNK,), jnp.int32),
        pltpu.VMEM((CHUNK,), jnp.float32),
        pltpu.VMEM((LANES,), jnp.float32),
        pltpu.SemaphoreType.DMA,
        pltpu.SemaphoreType.DMA,
        pltpu.SemaphoreType.DMA,
        pltpu.SemaphoreType.DMA,
    ],
)
def _sc_gather_loss(tags_hbm, flat_hbm, out_hbm,
                    lbl_v, idx_v, val_v, part_v, sem_a, sem_b, sem_c, sem_l):
    nc = 2
    wid = lax.axis_index("s") * nc + lax.axis_index("c")

    def lstage(m, c):
        pltpu.make_async_copy(tags_hbm.at[m, wid],
                              lbl_v.at[pl.ds(m * 8, 8), :], sem_l).start()
        return c

    lax.fori_loop(0, T // 8, lstage, 0)

    def lwait(m, c):
        pltpu.make_async_copy(tags_hbm.at[0, 0],
                              lbl_v.at[pl.ds(0, 8), :], sem_l).wait()
        return c

    lax.fori_loop(0, T // 8, lwait, 0)

    wbase = wid * 1024
    lane = lax.iota(jnp.int32, 16)

    def cbody(i, c):
        sl = pl.ds(i * LANES, LANES)
        q0 = (i & 7) << 4
        v = lbl_v[i >> 3, pl.ds(q0, LANES)]
        idx_v[sl] = ((v >> 3) << 15) + ((v & 7) << 7) + (wbase + q0 + lane)
        return c

    def compute(g):
        lax.fori_loop(g * GCH, (g + 1) * GCH, cbody, 0, unroll=4)

    def fire(g, sem):
        def fbody(t, c):
            sl = pl.ds(t * SW, SW)
            pltpu.make_async_copy(flat_hbm.at[idx_v.at[sl]], val_v.at[sl],
                                  sem).start()
            return c
        lax.fori_loop(g * G, (g + 1) * G, fbody, 0)

    def drain(g, sem):
        sl = pl.ds(g * GEL, GEL)
        pltpu.make_async_copy(flat_hbm.at[idx_v.at[sl]], val_v.at[sl],
                              sem).wait()

    def rbody(i, acc):
        sl = pl.ds(i * LANES, LANES)
        v = lbl_v[i >> 3, pl.ds((i & 7) << 4, LANES)]
        return acc + jnp.where(v != 0, val_v[sl], 0.0)

    def reduce(g, acc):
        return lax.fori_loop(g * GCH, (g + 1) * GCH, rbody, acc, unroll=4)

    def fused_body(gr, gc):
        def fbody(i, acc):
            slr = pl.ds(gr * GEL + i * LANES, LANES)
            ir = gr * GCH + i
            vr = lbl_v[ir >> 3, pl.ds((ir & 7) << 4, LANES)]
            acc = acc + jnp.where(vr != 0, val_v[slr], 0.0)
            ic = gc * GCH + i
            slc = pl.ds(gc * GEL + i * LANES, LANES)
            q0 = (ic & 7) << 4
            v = lbl_v[ic >> 3, pl.ds(q0, LANES)]
            idx_v[slc] = ((v >> 3) << 15) + ((v & 7) << 7) + (wbase + q0 + lane)
            return acc
        return fbody

    sems = (sem_a, sem_b, sem_c, sem_l)
    acc = jnp.zeros((LANES,), jnp.float32)
    for g in range(3):
        compute(g)
        fire(g, sems[g])
    for g in range(NG):
        drain(g, sems[g % 4])
        if g + 3 < NG:
            acc = lax.fori_loop(0, GCH, fused_body(g, g + 3), acc, unroll=4)
            fire(g + 3, sems[(g + 3) % 4])
        else:
            acc = reduce(g, acc)

    part_v[...] = acc * SCALE
    pltpu.sync_copy(part_v, out_hbm.at[wid])


def kernel(log_prob, tags_label):
    flat = (log_prob.T
            .reshape(V // 8, 8, B // 128, 128)
            .transpose(0, 2, 1, 3)
            .reshape(-1))
    tags = (tags_label.astype(jnp.int32).T
            .reshape(T // 8, 8, B // 128, 128)
            .transpose(0, 2, 1, 3))
    partials = _sc_gather_loss(tags, flat)
    return jnp.sum(partials)


# speedup vs baseline: 25.6512x; 1.0013x over previous
import functools

import jax
import jax.numpy as jnp
from jax import lax
from jax.experimental import pallas as pl
from jax.experimental.pallas import tpu as pltpu
from jax.experimental.pallas import tpu_sc as plsc

B = 4096
V = 100000
T = 200
NW = 32
CHUNK = (B * T) // NW
LANES = 16
SW = 128
G = 20
NG = CHUNK // (G * SW)
DEPTH = 4
GEL = G * SW
GCH = GEL // LANES
SCALE = -1.0 / float(B * T)


def _mesh():
    return plsc.VectorSubcoreMesh(core_axis_name="c", subcore_axis_name="s")


@functools.partial(
    pl.kernel,
    mesh=_mesh(),
    out_type=jax.ShapeDtypeStruct((NW, LANES), jnp.float32),
    scratch_types=[
        pltpu.VMEM((CHUNK // SW, SW), jnp.int32),
        pltpu.VMEM((CHUNK,), jnp.int32),
        pltpu.VMEM((CHUNK,), jnp.float32),
        pltpu.VMEM((LANES,), jnp.float32),
        pltpu.SemaphoreType.DMA,
        pltpu.SemaphoreType.DMA,
        pltpu.SemaphoreType.DMA,
        pltpu.SemaphoreType.DMA,
        pltpu.SemaphoreType.DMA,
    ],
)
def _sc_gather_loss(tags_hbm, flat_hbm, out_hbm,
                    lbl_v, idx_v, val_v, part_v,
                    sem_a, sem_b, sem_c, sem_d, sem_l):
    nc = 2
    wid = lax.axis_index("s") * nc + lax.axis_index("c")

    def lstage(m, c):
        pltpu.make_async_copy(tags_hbm.at[m, wid],
                              lbl_v.at[pl.ds(m * 8, 8), :], sem_l).start()
        return c

    lax.fori_loop(0, T // 8, lstage, 0)

    def lwait(m, c):
        pltpu.make_async_copy(tags_hbm.at[0, 0],
                              lbl_v.at[pl.ds(0, 8), :], sem_l).wait()
        return c

    lax.fori_loop(0, T // 8, lwait, 0)

    wbase = wid * 1024
    lane = lax.iota(jnp.int32, 16)

    def cbody(i, c):
        sl = pl.ds(i * LANES, LANES)
        q0 = (i & 7) << 4
        v = lbl_v[i >> 3, pl.ds(q0, LANES)]
        idx_v[sl] = ((v >> 3) << 15) + ((v & 7) << 7) + (wbase + q0 + lane)
        return c

    def compute(g):
        lax.fori_loop(g * GCH, (g + 1) * GCH, cbody, 0, unroll=4)

    def fire(g, sem):
        def fbody(t, c):
            sl = pl.ds(t * SW, SW)
            pltpu.make_async_copy(flat_hbm.at[idx_v.at[sl]], val_v.at[sl],
                                  sem).start()
            return c
        lax.fori_loop(g * G, (g + 1) * G, fbody, 0)

    def drain(g, sem):
        sl = pl.ds(g * GEL, GEL)
        pltpu.make_async_copy(flat_hbm.at[idx_v.at[sl]], val_v.at[sl],
                              sem).wait()

    def rbody(i, acc):
        sl = pl.ds(i * LANES, LANES)
        v = lbl_v[i >> 3, pl.ds((i & 7) << 4, LANES)]
        return acc + jnp.where(v != 0, val_v[sl], 0.0)

    def reduce(g, acc):
        return lax.fori_loop(g * GCH, (g + 1) * GCH, rbody, acc, unroll=4)

    def fused_body(gr, gc):
        def fbody(i, acc):
            slr = pl.ds(gr * GEL + i * LANES, LANES)
            ir = gr * GCH + i
            vr = lbl_v[ir >> 3, pl.ds((ir & 7) << 4, LANES)]
            acc = acc + jnp.where(vr != 0, val_v[slr], 0.0)
            ic = gc * GCH + i
            slc = pl.ds(gc * GEL + i * LANES, LANES)
            q0 = (ic & 7) << 4
            v = lbl_v[ic >> 3, pl.ds(q0, LANES)]
            idx_v[slc] = ((v >> 3) << 15) + ((v & 7) << 7) + (wbase + q0 + lane)
            return acc
        return fbody

    sems = (sem_a, sem_b, sem_c, sem_d, sem_l)
    nsem = len(sems)
    acc = jnp.zeros((LANES,), jnp.float32)
    for g in range(DEPTH):
        compute(g)
        fire(g, sems[g])
    for g in range(NG):
        drain(g, sems[g % nsem])
        if g + DEPTH < NG:
            acc = lax.fori_loop(0, GCH, fused_body(g, g + DEPTH), acc,
                                unroll=4)
            fire(g + DEPTH, sems[(g + DEPTH) % nsem])
        else:
            acc = reduce(g, acc)

    part_v[...] = acc * SCALE
    pltpu.sync_copy(part_v, out_hbm.at[wid])


def kernel(log_prob, tags_label):
    flat = (log_prob.T
            .reshape(V // 8, 8, B // 128, 128)
            .transpose(0, 2, 1, 3)
            .reshape(-1))
    tags = (tags_label.astype(jnp.int32).T
            .reshape(T // 8, 8, B // 128, 128)
            .transpose(0, 2, 1, 3))
    partials = _sc_gather_loss(tags, flat)
    return jnp.sum(partials)
